# Initial kernel scaffold; baseline (speedup 1.0000x reference)
import jax
import jax.numpy as jnp
from jax.experimental import pallas as pl


def kernel(input, table, conv_w, conv_b, W_proj, b_proj, W_gate, b_gate):
    raise NotImplementedError("write your pallas kernel here")


# speedup vs baseline: 17.1383x; 17.1383x over previous
import functools

import jax
import jax.numpy as jnp
from jax.experimental import pallas as pl
from jax.experimental.pallas import tpu as pltpu

S, B, W = 50, 1024, 21
V, C, E = 96, 50, 128
K = 5
T = W - K + 1
N = S * B


def _fold_body(table_ref, cw_ref, cb_ref, out_ref):
    tab = table_ref[...]
    for k in range(K):
        wk = cw_ref[k]
        mk = jax.lax.dot_general(tab, wk, (((1,), (0,)), ((), ())),
                                 preferred_element_type=jnp.float32)
        if k == 0:
            mk = mk + cb_ref[...]
        out_ref[:, k * E:(k + 1) * E] = mk


def _fold_tables(table, conv_w, conv_b):
    cw = jnp.transpose(conv_w, (2, 1, 0))
    cb = conv_b.reshape(1, E)
    return pl.pallas_call(
        _fold_body,
        out_shape=jax.ShapeDtypeStruct((V, K * E), jnp.float32),
    )(table, cw, cb)


def _main_body(idx_ref, m_ref, wp_ref, bp_ref, wg_ref, bg_ref, out_ref, *, n):
    idx = idx_ref[...]
    iota = jax.lax.broadcasted_iota(jnp.int32, (n * W, V), 1)
    oh = (idx == iota).astype(jnp.float32)
    z = jax.lax.dot_general(oh, m_ref[...], (((1,), (0,)), ((), ())),
                            preferred_element_type=jnp.float32)
    z = z.reshape(n, W, K * E)
    conv = z[:, 0:T, 0:E]
    for k in range(1, K):
        conv = conv + z[:, k:k + T, k * E:(k + 1) * E]
    m = jnp.maximum(jnp.max(conv, axis=1), 0.0)
    proj = jax.lax.dot_general(m, wp_ref[...], (((1,), (1,)), ((), ())),
                               preferred_element_type=jnp.float32)
    proj = jnp.maximum(proj + bp_ref[...], 0.0)
    gate = jax.lax.dot_general(m, wg_ref[...], (((1,), (1,)), ((), ())),
                               preferred_element_type=jnp.float32)
    gate = jax.nn.sigmoid(gate + bg_ref[...])
    out_ref[...] = gate * proj + (1.0 - gate) * m


def _run_main(idxw, mcat, w_proj, b_proj, w_gate, b_gate, n):
    grid = (N // n,)
    return pl.pallas_call(
        functools.partial(_main_body, n=n),
        grid=grid,
        in_specs=[
            pl.BlockSpec((n * W, 1), lambda i: (i, 0)),
            pl.BlockSpec((V, K * E), lambda i: (0, 0)),
            pl.BlockSpec((E, E), lambda i: (0, 0)),
            pl.BlockSpec((1, E), lambda i: (0, 0)),
            pl.BlockSpec((E, E), lambda i: (0, 0)),
            pl.BlockSpec((1, E), lambda i: (0, 0)),
        ],
        out_specs=pl.BlockSpec((n, E), lambda i: (i, 0)),
        out_shape=jax.ShapeDtypeStruct((N, E), jnp.float32),
        compiler_params=pltpu.CompilerParams(
            dimension_semantics=("arbitrary",),
        ),
    )(idxw.reshape(N * W, 1), mcat, w_proj, b_proj.reshape(1, E),
      w_gate, b_gate.reshape(1, E))


def kernel(input, table, conv_w, conv_b, W_proj, b_proj, W_gate, b_gate):
    idxw = jnp.transpose(input, (1, 0, 2)).reshape(N, W).astype(jnp.int32)
    mcat = _fold_tables(table, conv_w, conv_b)
    out = _run_main(idxw, mcat, W_proj, b_proj, W_gate, b_gate, n=256)
    return out.reshape(S, B, E)


# speedup vs baseline: 36.4735x; 2.1282x over previous
import functools

import jax
import jax.numpy as jnp
from jax import lax
from jax.experimental import pallas as pl
from jax.experimental.pallas import tpu as pltpu
from jax.experimental.pallas import tpu_sc as plsc

S, B, W = 50, 1024, 21
V, C, E = 96, 50, 128
K = 5
T = W - K + 1
N = S * B
L = 16
NCHUNK = E // L

NC, NS = 2, 16
NW = NC * NS
WPW = N // NW
CHUNK = 160
NLOOPS = WPW // CHUNK


def _fold_body(table_ref, cw_ref, cb_ref, out_ref):
    tab = table_ref[...]
    for k in range(K):
        wk = cw_ref[k]
        mk = jax.lax.dot_general(tab, wk, (((1,), (0,)), ((), ())),
                                 preferred_element_type=jnp.float32)
        if k == 0:
            mk = mk + cb_ref[...]
        out_ref[k * V:(k + 1) * V, :] = mk


def _fold_tables(table, conv_w, conv_b):
    cw = jnp.transpose(conv_w, (2, 1, 0))
    cb = conv_b.reshape(1, E)
    return pl.pallas_call(
        _fold_body,
        out_shape=jax.ShapeDtypeStruct((K * V, E), jnp.float32),
    )(table, cw, cb)


def _sc_conv_body(m_hbm, idx_hbm, out_hbm, m_v, idx_v, out_v):
    wid = lax.axis_index("s") * NC + lax.axis_index("c")
    base = wid * WPW
    pltpu.sync_copy(m_hbm, m_v)
    iota = lax.iota(jnp.int32, L)

    def chunk_body(ci, _):
        w0 = base + ci * CHUNK
        pltpu.sync_copy(idx_hbm.at[pl.ds(w0 * W, CHUNK * W)], idx_v)

        def word_body(wl, _):
            woff = wl * W
            cj = [plsc.load_gather(
                      idx_v, [jnp.broadcast_to(woff + j, (L,))]) * E
                  for j in range(W)]
            run = [jnp.zeros((L,), jnp.float32) for _ in range(NCHUNK)]
            for t in range(T):
                idxk = [cj[t + k] + (k * V * E) + iota for k in range(K)]
                for ch in range(NCHUNK):
                    acc = plsc.load_gather(m_v, [idxk[0]])
                    for k in range(1, K):
                        acc = acc + plsc.load_gather(m_v, [idxk[k]])
                    run[ch] = jnp.maximum(run[ch], acc)
                    if ch + 1 < NCHUNK:
                        for k in range(K):
                            idxk[k] = idxk[k] + L
            for ch in range(NCHUNK):
                out_v[wl, pl.ds(ch * L, L)] = run[ch]
            return ()

        lax.fori_loop(0, CHUNK, word_body, ())
        pltpu.sync_copy(out_v, out_hbm.at[pl.ds(w0, CHUNK)])
        return ()

    lax.fori_loop(0, NLOOPS, chunk_body, ())


def _sc_conv(mcat, idxw):
    mesh = plsc.VectorSubcoreMesh(core_axis_name="c", subcore_axis_name="s")
    f = functools.partial(
        pl.kernel, mesh=mesh,
        out_type=jax.ShapeDtypeStruct((N, E), jnp.float32),
        scratch_types=[
            pltpu.VMEM((K * V * E,), jnp.float32),
            pltpu.VMEM((CHUNK * W,), jnp.int32),
            pltpu.VMEM((CHUNK, E), jnp.float32),
        ],
        compiler_params=pltpu.CompilerParams(needs_layout_passes=False),
    )(_sc_conv_body)
    return f(mcat.reshape(K * V * E), idxw.reshape(N * W))


def _hw_body(x_ref, wp_ref, bp_ref, wg_ref, bg_ref, out_ref):
    x = x_ref[...]
    proj = jax.lax.dot_general(x, wp_ref[...], (((1,), (1,)), ((), ())),
                               preferred_element_type=jnp.float32)
    proj = jnp.maximum(proj + bp_ref[...], 0.0)
    gate = jax.lax.dot_general(x, wg_ref[...], (((1,), (1,)), ((), ())),
                               preferred_element_type=jnp.float32)
    gate = jax.nn.sigmoid(gate + bg_ref[...])
    out_ref[...] = gate * proj + (1.0 - gate) * x


def _highway(x, w_proj, b_proj, w_gate, b_gate, n=2048):
    return pl.pallas_call(
        _hw_body,
        grid=(N // n,),
        in_specs=[
            pl.BlockSpec((n, E), lambda i: (i, 0)),
            pl.BlockSpec((E, E), lambda i: (0, 0)),
            pl.BlockSpec((1, E), lambda i: (0, 0)),
            pl.BlockSpec((E, E), lambda i: (0, 0)),
            pl.BlockSpec((1, E), lambda i: (0, 0)),
        ],
        out_specs=pl.BlockSpec((n, E), lambda i: (i, 0)),
        out_shape=jax.ShapeDtypeStruct((N, E), jnp.float32),
        compiler_params=pltpu.CompilerParams(
            dimension_semantics=("arbitrary",),
        ),
    )(x, w_proj, b_proj.reshape(1, E), w_gate, b_gate.reshape(1, E))


def kernel(input, table, conv_w, conv_b, W_proj, b_proj, W_gate, b_gate):
    idxw = jnp.transpose(input, (1, 0, 2)).reshape(N, W).astype(jnp.int32)
    mcat = _fold_tables(table, conv_w, conv_b)
    conv = _sc_conv(mcat, idxw)
    out = _highway(conv, W_proj, b_proj, W_gate, b_gate)
    return out.reshape(S, B, E)
